# baseline (speedup 1.0000x reference)
import functools

import jax
import jax.numpy as jnp
from jax import lax
from jax.experimental import pallas as pl
from jax.experimental.pallas import tpu as pltpu
from jax.experimental.pallas import tpu_sc as plsc

_NW = 32
_LANES = 16
_CHUNK = 40
_EPS = 1e-5


def _rsqrt_vec(x):
  bits = lax.bitcast_convert_type(x, jnp.int32)
  y = lax.bitcast_convert_type(jnp.int32(0x5F3759DF) - (bits >> 1),
                               jnp.float32)
  half = x * 0.5
  for _ in range(3):
    y = y * (1.5 - half * y * y)
  return y


def _make_sc_encoder(n_tok, seq_len, hid, vocab):
  tpw = n_tok // _NW
  n_chunks = tpw // _CHUNK
  nvec = hid // _LANES
  mesh = plsc.VectorSubcoreMesh(core_axis_name="c", subcore_axis_name="s")

  @functools.partial(
      pl.kernel,
      mesh=mesh,
      out_type=jax.ShapeDtypeStruct((n_tok, hid), jnp.float32),
      scratch_types=[
          pltpu.VMEM((tpw,), jnp.int32),
          pltpu.VMEM((seq_len, hid), jnp.float32),
          pltpu.VMEM((hid,), jnp.float32),
          pltpu.VMEM((hid,), jnp.float32),
          pltpu.VMEM((_CHUNK, hid), jnp.float32),
          pltpu.VMEM((32,), jnp.float32),
          pltpu.VMEM((32,), jnp.float32),
          pltpu.SemaphoreType.DMA,
      ],
  )
  def enc(ids_hbm, table_hbm, pos_hbm, w_hbm, b_hbm, out_hbm,
          idx_v, pos_v, w_v, b_v, rows_v, red_a, red_b, sem):
    wid = lax.axis_index("s") * 2 + lax.axis_index("c")
    base = wid * tpw

    pltpu.sync_copy(ids_hbm.at[pl.ds(base, tpw)], idx_v)
    pltpu.sync_copy(pos_hbm, pos_v)
    pltpu.sync_copy(w_hbm, w_v)
    pltpu.sync_copy(b_hbm, b_v)

    def do_chunk(c, _):
      off = c * _CHUNK
      pltpu.async_copy(
          table_hbm.at[idx_v.at[pl.ds(off, _CHUNK)]], rows_v, sem).wait()

      def do_row(r, _):
        tok = off + r
        l = lax.rem(tok, seq_len)

        def pass1(j, carry):
          sv, ssv = carry
          sl = pl.ds(j * _LANES, _LANES)
          v = rows_v[r, sl] + pos_v[l, sl]
          v = jnp.minimum(jnp.maximum(v, -10.0), 10.0)
          rows_v[r, sl] = v
          return sv + v, ssv + v * v

        zero = jnp.zeros((_LANES,), jnp.float32)
        sv, ssv = lax.fori_loop(0, nvec, pass1, (zero, zero))

        def lane_total(v, red):
          for sh in (8, 4, 2, 1):
            red[pl.ds(0, _LANES)] = v
            red[pl.ds(_LANES, _LANES)] = v
            v = v + red[pl.ds(sh, _LANES)]
          return v

        inv_n = jnp.float32(1.0 / hid)
        mu = lane_total(sv, red_a) * inv_n
        ex2 = lane_total(ssv, red_b) * inv_n
        var = ex2 - mu * mu
        rstd = _rsqrt_vec(var + _EPS)
        shift = -(mu * rstd)

        def pass2(j, _):
          sl = pl.ds(j * _LANES, _LANES)
          v = rows_v[r, sl]
          y = (v * rstd + shift) * w_v[sl] + b_v[sl]
          y = jnp.minimum(jnp.maximum(y, -50.0), 50.0)
          rows_v[r, sl] = y
          return 0

        lax.fori_loop(0, nvec, pass2, 0)
        return 0

      lax.fori_loop(0, _CHUNK, do_row, 0)
      pltpu.sync_copy(rows_v, out_hbm.at[pl.ds(base + off, _CHUNK)])
      return 0

    lax.fori_loop(0, n_chunks, do_chunk, 0)

  return enc


def kernel(input_ids, attention_mask, token_embedding, pos_emb, ln_w, ln_b):
  b, l = input_ids.shape
  vocab, hid = token_embedding.shape
  n_tok = b * l
  ids = jnp.clip(input_ids.reshape(n_tok).astype(jnp.int32), 0, vocab - 1)
  pos = pos_emb[0, :l, :]
  enc = _make_sc_encoder(n_tok, l, hid, vocab)
  out = enc(ids, token_embedding, pos, ln_w.astype(jnp.float32),
            ln_b.astype(jnp.float32))
  out = out * attention_mask.reshape(n_tok, 1)
  return out.reshape(b, l, hid)
